# Initial kernel scaffold; baseline (speedup 1.0000x reference)
import jax
import jax.numpy as jnp
from jax.experimental import pallas as pl


def kernel(x, h, c, edge_index, W_i, b_i, W_f, b_f, W_o, b_o, W_g, b_g):
    raise NotImplementedError("write your pallas kernel here")


# speedup vs baseline: 16.6390x; 16.6390x over previous
import functools

import jax
import jax.numpy as jnp
from jax import lax
from jax.experimental import pallas as pl
from jax.experimental.pallas import tpu as pltpu
from jax.experimental.pallas import tpu_sc as plsc

N = 10000
FH = 128
NC = 2
NS = 16
CHUNK = 128
N_PAD = 10112
RPS = N_PAD // NS
TRASH = N


def _deg_body(rows_per_worker, dstm, z128, o128, deg0, deg1,
              acc, dst_v, ones_v):
    c = lax.axis_index("c")
    s = lax.axis_index("s")
    pltpu.sync_copy(z128, acc.at[pl.ds(s * RPS, RPS)])
    pltpu.sync_copy(o128, ones_v)
    wid = c * NS + s
    pltpu.sync_copy(dstm.at[pl.ds(wid * rows_per_worker, rows_per_worker)],
                    dst_v)
    plsc.subcore_barrier()

    def body(k, carry):
        pltpu.sync_copy(ones_v, acc.at[dst_v.at[k]], add=True)
        return carry

    lax.fori_loop(0, rows_per_worker, body, 0)
    plsc.subcore_barrier()

    @pl.when(c == 0)
    def _():
        pltpu.sync_copy(acc.at[pl.ds(s * RPS, RPS)],
                        deg0.at[pl.ds(s * RPS, RPS)])

    @pl.when(c == 1)
    def _():
        pltpu.sync_copy(acc.at[pl.ds(s * RPS, RPS)],
                        deg1.at[pl.ds(s * RPS, RPS)])


def _deg_call(dstm, z128, o128, rows_all):
    rows_per_worker = rows_all // (NC * NS)
    body = functools.partial(_deg_body, rows_per_worker)
    f = pl.kernel(
        body,
        out_type=(
            jax.ShapeDtypeStruct((N_PAD, FH), jnp.float32),
            jax.ShapeDtypeStruct((N_PAD, FH), jnp.float32),
        ),
        mesh=plsc.VectorSubcoreMesh(core_axis_name="c", subcore_axis_name="s"),
        scratch_types=(
            pltpu.VMEM_SHARED((N_PAD, FH), jnp.float32),
            pltpu.VMEM((rows_per_worker, CHUNK), jnp.int32),
            pltpu.VMEM((CHUNK, FH), jnp.float32),
        ),
    )
    return f(dstm, z128, o128)


IDXB = 16


def _agg_body(rows_per_sub, cs0, cs1, srcm, dstm, z128, agg0, agg1,
              acc, src_v, dst_v, rows_a, rows_b, sem_a, sem_b):
    c = lax.axis_index("c")
    s = lax.axis_index("s")
    pltpu.sync_copy(z128, acc.at[pl.ds(s * RPS, RPS)])
    base = s * rows_per_sub
    plsc.subcore_barrier()

    def run(cs_hbm, agg_hbm):
        def blk(b, carry):
            pltpu.sync_copy(srcm.at[pl.ds(base + b * IDXB, IDXB)], src_v)
            pltpu.sync_copy(dstm.at[pl.ds(base + b * IDXB, IDXB)], dst_v)

            def pair(j, carry2):
                k0 = 2 * j
                k1 = k0 + 1
                d0 = pltpu.async_copy(cs_hbm.at[src_v.at[k0]], rows_a, sem_a)
                d1 = pltpu.async_copy(cs_hbm.at[src_v.at[k1]], rows_b, sem_b)
                d0.wait()
                pltpu.sync_copy(rows_a, acc.at[dst_v.at[k0]], add=True)
                d1.wait()
                pltpu.sync_copy(rows_b, acc.at[dst_v.at[k1]], add=True)
                return carry2

            return lax.fori_loop(0, IDXB // 2, pair, carry)

        lax.fori_loop(0, rows_per_sub // IDXB, blk, 0)
        plsc.subcore_barrier()
        pltpu.sync_copy(acc.at[pl.ds(s * RPS, RPS)],
                        agg_hbm.at[pl.ds(s * RPS, RPS)])

    @pl.when(c == 0)
    def _():
        run(cs0, agg0)

    @pl.when(c == 1)
    def _():
        run(cs1, agg1)


def _agg_call(cs0, cs1, srcm, dstm, z128, rows_all):
    rows_per_sub = rows_all // NS
    body = functools.partial(_agg_body, rows_per_sub)
    f = pl.kernel(
        body,
        out_type=(
            jax.ShapeDtypeStruct((N_PAD, FH), jnp.float32),
            jax.ShapeDtypeStruct((N_PAD, FH), jnp.float32),
        ),
        mesh=plsc.VectorSubcoreMesh(core_axis_name="c", subcore_axis_name="s"),
        scratch_types=(
            pltpu.VMEM_SHARED((N_PAD, FH), jnp.float32),
            pltpu.VMEM((IDXB, CHUNK), jnp.int32),
            pltpu.VMEM((IDXB, CHUNK), jnp.int32),
            pltpu.VMEM((CHUNK, FH), jnp.float32),
            pltpu.VMEM((CHUNK, FH), jnp.float32),
            pltpu.SemaphoreType.DMA,
            pltpu.SemaphoreType.DMA,
        ),
    )
    return f(cs0, cs1, srcm, dstm, z128)


def _scale_body(x_ref, h_ref, d0_ref, d1_ref, cs0_ref, cs1_ref):
    deg = 1.0 + d0_ref[:, 0:1] + d1_ref[:, 0:1]
    dinv = lax.rsqrt(deg)
    cs0_ref[...] = x_ref[...] * dinv
    cs1_ref[...] = h_ref[...] * dinv


def _scale_call(x, h, deg0, deg1):
    B = 400
    grid = N // B
    return pl.pallas_call(
        _scale_body,
        grid=(grid,),
        in_specs=[
            pl.BlockSpec((B, FH), lambda i: (i, 0)),
            pl.BlockSpec((B, FH), lambda i: (i, 0)),
            pl.BlockSpec((B, FH), lambda i: (i, 0)),
            pl.BlockSpec((B, FH), lambda i: (i, 0)),
        ],
        out_specs=[
            pl.BlockSpec((B, FH), lambda i: (i, 0)),
            pl.BlockSpec((B, FH), lambda i: (i, 0)),
        ],
        out_shape=[
            jax.ShapeDtypeStruct((N, FH), jnp.float32),
            jax.ShapeDtypeStruct((N, FH), jnp.float32),
        ],
    )(x, h, deg0, deg1)


def _gate_body(a0_ref, a1_ref, cs0_ref, cs1_ref, d0_ref, d1_ref, c_ref,
               w_ref, b_ref, h_out, c_out):
    dinv = lax.rsqrt(1.0 + d0_ref[:, 0:1] + d1_ref[:, 0:1])
    z0 = (a0_ref[...] + cs0_ref[...]) * dinv
    z1 = (a1_ref[...] + cs1_ref[...]) * dinv
    z = jnp.concatenate([z0, z1], axis=1)
    g4 = jnp.dot(z, w_ref[...], preferred_element_type=jnp.float32)
    g4 = g4 + b_ref[...]
    gi = jax.nn.sigmoid(g4[:, 0 * FH:1 * FH])
    gf = jax.nn.sigmoid(g4[:, 1 * FH:2 * FH])
    go = jax.nn.sigmoid(g4[:, 2 * FH:3 * FH])
    gg = jnp.tanh(g4[:, 3 * FH:4 * FH])
    cn = gf * c_ref[...] + gi * gg
    h_out[...] = go * jnp.tanh(cn)
    c_out[...] = cn


def _gate_call(agg0, agg1, cs0, cs1, deg0, deg1, c, w4, b4):
    B = 400
    grid = N // B
    return pl.pallas_call(
        _gate_body,
        grid=(grid,),
        in_specs=[
            pl.BlockSpec((B, FH), lambda i: (i, 0)),
            pl.BlockSpec((B, FH), lambda i: (i, 0)),
            pl.BlockSpec((B, FH), lambda i: (i, 0)),
            pl.BlockSpec((B, FH), lambda i: (i, 0)),
            pl.BlockSpec((B, FH), lambda i: (i, 0)),
            pl.BlockSpec((B, FH), lambda i: (i, 0)),
            pl.BlockSpec((B, FH), lambda i: (i, 0)),
            pl.BlockSpec((2 * FH, 4 * FH), lambda i: (0, 0)),
            pl.BlockSpec((1, 4 * FH), lambda i: (0, 0)),
        ],
        out_specs=[
            pl.BlockSpec((B, FH), lambda i: (i, 0)),
            pl.BlockSpec((B, FH), lambda i: (i, 0)),
        ],
        out_shape=[
            jax.ShapeDtypeStruct((N, FH), jnp.float32),
            jax.ShapeDtypeStruct((N, FH), jnp.float32),
        ],
    )(agg0, agg1, cs0, cs1, deg0, deg1, c, w4, b4)


def kernel(x, h, c, edge_index, W_i, b_i, W_f, b_f, W_o, b_o, W_g, b_g):
    e = edge_index.shape[1]
    group = NC * NS * CHUNK * 8
    e_pad = ((e + group - 1) // group) * group
    rows_all = e_pad // CHUNK

    src = edge_index[0].astype(jnp.int32)
    dst = edge_index[1].astype(jnp.int32)
    pad = e_pad - e
    srcm = jnp.concatenate([src, jnp.zeros((pad,), jnp.int32)])
    srcm = srcm.reshape(rows_all, CHUNK)
    dstm = jnp.concatenate([dst, jnp.full((pad,), TRASH, jnp.int32)])
    dstm = dstm.reshape(rows_all, CHUNK)

    z128 = jnp.zeros((RPS, FH), jnp.float32)
    o128 = jnp.ones((CHUNK, FH), jnp.float32)

    deg0, deg1 = _deg_call(dstm, z128, o128, rows_all)
    cs0, cs1 = _scale_call(x, h, deg0, deg1)
    agg0, agg1 = _agg_call(cs0, cs1, srcm, dstm, z128, rows_all)

    w4 = jnp.concatenate([W_i, W_f, W_o, W_g], axis=1)
    b4 = jnp.concatenate([b_i, b_f, b_o, b_g]).reshape(1, 4 * FH)
    h_new, c_new = _gate_call(agg0, agg1, cs0, cs1, deg0, deg1, c, w4, b4)
    return (h_new, c_new)


# speedup vs baseline: 17.9505x; 1.0788x over previous
import functools

import jax
import jax.numpy as jnp
from jax import lax
from jax.experimental import pallas as pl
from jax.experimental.pallas import tpu as pltpu
from jax.experimental.pallas import tpu_sc as plsc

N = 10000
FH = 128
NC = 2
NS = 16
CHUNK = 128
N_PAD = 10112
RPS = N_PAD // NS
TRASH = N


def _deg_body(rows_per_worker, dstm, z128, o128, deg0, deg1,
              acc, dst_v, ones_v):
    c = lax.axis_index("c")
    s = lax.axis_index("s")
    pltpu.sync_copy(z128, acc.at[pl.ds(s * RPS, RPS)])
    pltpu.sync_copy(o128, ones_v)
    wid = c * NS + s
    pltpu.sync_copy(dstm.at[pl.ds(wid * rows_per_worker, rows_per_worker)],
                    dst_v)
    plsc.subcore_barrier()

    def body(k, carry):
        pltpu.sync_copy(ones_v, acc.at[dst_v.at[k]], add=True)
        return carry

    lax.fori_loop(0, rows_per_worker, body, 0)
    plsc.subcore_barrier()

    @pl.when(c == 0)
    def _():
        pltpu.sync_copy(acc.at[pl.ds(s * RPS, RPS)],
                        deg0.at[pl.ds(s * RPS, RPS)])

    @pl.when(c == 1)
    def _():
        pltpu.sync_copy(acc.at[pl.ds(s * RPS, RPS)],
                        deg1.at[pl.ds(s * RPS, RPS)])


def _deg_call(dstm, z128, o128, rows_all):
    rows_per_worker = rows_all // (NC * NS)
    body = functools.partial(_deg_body, rows_per_worker)
    f = pl.kernel(
        body,
        out_type=(
            jax.ShapeDtypeStruct((N_PAD, FH), jnp.float32),
            jax.ShapeDtypeStruct((N_PAD, FH), jnp.float32),
        ),
        mesh=plsc.VectorSubcoreMesh(core_axis_name="c", subcore_axis_name="s"),
        scratch_types=(
            pltpu.VMEM_SHARED((N_PAD, FH), jnp.float32),
            pltpu.VMEM((rows_per_worker, CHUNK), jnp.int32),
            pltpu.VMEM((CHUNK, FH), jnp.float32),
        ),
    )
    return f(dstm, z128, o128)


SEG = 40


def _agg_body(rows_per_sub, cs0, cs1, srcm, dstm, z128, agg0, agg1,
              acc, src_v, dst_v, rows_a, rows_b,
              sem_ga, sem_gb, sem_sa, sem_sb):
    c = lax.axis_index("c")
    s = lax.axis_index("s")
    pltpu.sync_copy(z128, acc.at[pl.ds(s * RPS, RPS)])
    base = s * rows_per_sub
    plsc.subcore_barrier()

    rows = (rows_a, rows_b)
    sem_g = (sem_ga, sem_gb)
    sem_s = (sem_sa, sem_sb)

    def run(cs_hbm, agg_hbm):
        def gather(b, k):
            pltpu.async_copy(cs_hbm.at[src_v.at[k]], rows[b], sem_g[b])

        def wait_gather(b):
            pltpu.make_async_copy(cs_hbm.at[src_v.at[0]], rows[b],
                                  sem_g[b]).wait()

        def scatter(b, k):
            pltpu.async_copy(rows[b], acc.at[dst_v.at[k]], sem_s[b],
                             add=True)

        def wait_scatter(b):
            pltpu.make_async_copy(rows[b], acc.at[dst_v.at[0]],
                                  sem_s[b]).wait()

        def seg_body(g, carry):
            pltpu.sync_copy(srcm.at[pl.ds(base + g * SEG, SEG)], src_v)
            pltpu.sync_copy(dstm.at[pl.ds(base + g * SEG, SEG)], dst_v)
            gather(0, 0)

            def pair(q, carry2):
                k0 = 2 * q
                wait_gather(0)
                scatter(0, k0)

                @pl.when(q > 0)
                def _():
                    wait_scatter(1)

                gather(1, k0 + 1)
                wait_gather(1)
                scatter(1, k0 + 1)
                wait_scatter(0)

                @pl.when(q < SEG // 2 - 1)
                def _():
                    gather(0, k0 + 2)

                return carry2

            lax.fori_loop(0, SEG // 2, pair, 0)
            wait_scatter(1)
            return carry

        lax.fori_loop(0, rows_per_sub // SEG, seg_body, 0)
        plsc.subcore_barrier()
        pltpu.sync_copy(acc.at[pl.ds(s * RPS, RPS)],
                        agg_hbm.at[pl.ds(s * RPS, RPS)])

    @pl.when(c == 0)
    def _():
        run(cs0, agg0)

    @pl.when(c == 1)
    def _():
        run(cs1, agg1)


def _agg_call(cs0, cs1, srcm, dstm, z128, rows_all):
    rows_per_sub = rows_all // NS
    body = functools.partial(_agg_body, rows_per_sub)
    f = pl.kernel(
        body,
        out_type=(
            jax.ShapeDtypeStruct((N_PAD, FH), jnp.float32),
            jax.ShapeDtypeStruct((N_PAD, FH), jnp.float32),
        ),
        mesh=plsc.VectorSubcoreMesh(core_axis_name="c", subcore_axis_name="s"),
        scratch_types=(
            pltpu.VMEM_SHARED((N_PAD, FH), jnp.float32),
            pltpu.VMEM((SEG, CHUNK), jnp.int32),
            pltpu.VMEM((SEG, CHUNK), jnp.int32),
            pltpu.VMEM((CHUNK, FH), jnp.float32),
            pltpu.VMEM((CHUNK, FH), jnp.float32),
            pltpu.SemaphoreType.DMA,
            pltpu.SemaphoreType.DMA,
            pltpu.SemaphoreType.DMA,
            pltpu.SemaphoreType.DMA,
        ),
    )
    return f(cs0, cs1, srcm, dstm, z128)


def _scale_body(x_ref, h_ref, d0_ref, d1_ref, cs0_ref, cs1_ref):
    deg = 1.0 + d0_ref[:, 0:1] + d1_ref[:, 0:1]
    dinv = lax.rsqrt(deg)
    cs0_ref[...] = x_ref[...] * dinv
    cs1_ref[...] = h_ref[...] * dinv


def _scale_call(x, h, deg0, deg1):
    B = 400
    grid = N // B
    return pl.pallas_call(
        _scale_body,
        grid=(grid,),
        in_specs=[
            pl.BlockSpec((B, FH), lambda i: (i, 0)),
            pl.BlockSpec((B, FH), lambda i: (i, 0)),
            pl.BlockSpec((B, FH), lambda i: (i, 0)),
            pl.BlockSpec((B, FH), lambda i: (i, 0)),
        ],
        out_specs=[
            pl.BlockSpec((B, FH), lambda i: (i, 0)),
            pl.BlockSpec((B, FH), lambda i: (i, 0)),
        ],
        out_shape=[
            jax.ShapeDtypeStruct((N, FH), jnp.float32),
            jax.ShapeDtypeStruct((N, FH), jnp.float32),
        ],
    )(x, h, deg0, deg1)


def _gate_body(a0_ref, a1_ref, cs0_ref, cs1_ref, d0_ref, d1_ref, c_ref,
               w_ref, b_ref, h_out, c_out):
    dinv = lax.rsqrt(1.0 + d0_ref[:, 0:1] + d1_ref[:, 0:1])
    z0 = (a0_ref[...] + cs0_ref[...]) * dinv
    z1 = (a1_ref[...] + cs1_ref[...]) * dinv
    z = jnp.concatenate([z0, z1], axis=1)
    g4 = jnp.dot(z, w_ref[...], preferred_element_type=jnp.float32)
    g4 = g4 + b_ref[...]
    gi = jax.nn.sigmoid(g4[:, 0 * FH:1 * FH])
    gf = jax.nn.sigmoid(g4[:, 1 * FH:2 * FH])
    go = jax.nn.sigmoid(g4[:, 2 * FH:3 * FH])
    gg = jnp.tanh(g4[:, 3 * FH:4 * FH])
    cn = gf * c_ref[...] + gi * gg
    h_out[...] = go * jnp.tanh(cn)
    c_out[...] = cn


def _gate_call(agg0, agg1, cs0, cs1, deg0, deg1, c, w4, b4):
    B = 400
    grid = N // B
    return pl.pallas_call(
        _gate_body,
        grid=(grid,),
        in_specs=[
            pl.BlockSpec((B, FH), lambda i: (i, 0)),
            pl.BlockSpec((B, FH), lambda i: (i, 0)),
            pl.BlockSpec((B, FH), lambda i: (i, 0)),
            pl.BlockSpec((B, FH), lambda i: (i, 0)),
            pl.BlockSpec((B, FH), lambda i: (i, 0)),
            pl.BlockSpec((B, FH), lambda i: (i, 0)),
            pl.BlockSpec((B, FH), lambda i: (i, 0)),
            pl.BlockSpec((2 * FH, 4 * FH), lambda i: (0, 0)),
            pl.BlockSpec((1, 4 * FH), lambda i: (0, 0)),
        ],
        out_specs=[
            pl.BlockSpec((B, FH), lambda i: (i, 0)),
            pl.BlockSpec((B, FH), lambda i: (i, 0)),
        ],
        out_shape=[
            jax.ShapeDtypeStruct((N, FH), jnp.float32),
            jax.ShapeDtypeStruct((N, FH), jnp.float32),
        ],
    )(agg0, agg1, cs0, cs1, deg0, deg1, c, w4, b4)


def kernel(x, h, c, edge_index, W_i, b_i, W_f, b_f, W_o, b_o, W_g, b_g):
    e = edge_index.shape[1]
    group = NC * NS * CHUNK * 8
    e_pad = ((e + group - 1) // group) * group
    rows_all = e_pad // CHUNK

    src = edge_index[0].astype(jnp.int32)
    dst = edge_index[1].astype(jnp.int32)
    pad = e_pad - e
    srcm = jnp.concatenate([src, jnp.zeros((pad,), jnp.int32)])
    srcm = srcm.reshape(rows_all, CHUNK)
    dstm = jnp.concatenate([dst, jnp.full((pad,), TRASH, jnp.int32)])
    dstm = dstm.reshape(rows_all, CHUNK)

    z128 = jnp.zeros((RPS, FH), jnp.float32)
    o128 = jnp.ones((CHUNK, FH), jnp.float32)

    deg0, deg1 = _deg_call(dstm, z128, o128, rows_all)
    cs0, cs1 = _scale_call(x, h, deg0, deg1)
    agg0, agg1 = _agg_call(cs0, cs1, srcm, dstm, z128, rows_all)

    w4 = jnp.concatenate([W_i, W_f, W_o, W_g], axis=1)
    b4 = jnp.concatenate([b_i, b_f, b_o, b_g]).reshape(1, 4 * FH)
    h_new, c_new = _gate_call(agg0, agg1, cs0, cs1, deg0, deg1, c, w4, b4)
    return (h_new, c_new)


# speedup vs baseline: 18.7075x; 1.0422x over previous
import functools

import jax
import jax.numpy as jnp
from jax import lax
from jax.experimental import pallas as pl
from jax.experimental.pallas import tpu as pltpu
from jax.experimental.pallas import tpu_sc as plsc

N = 10000
FH = 128
NC = 2
NS = 16
CHUNK = 128
N_PAD = 10112
RPS = N_PAD // NS
TRASH = N


def _deg_body(rows_per_worker, dstm, z128, o128, deg0, deg1,
              acc, dst_v, ones_v, sem):
    c = lax.axis_index("c")
    s = lax.axis_index("s")
    pltpu.sync_copy(z128, acc.at[pl.ds(s * RPS, RPS)])
    pltpu.sync_copy(o128, ones_v)
    wid = c * NS + s
    pltpu.sync_copy(dstm.at[pl.ds(wid * rows_per_worker, rows_per_worker)],
                    dst_v)
    plsc.subcore_barrier()

    def grp(g, carry):
        for j in range(8):
            pltpu.async_copy(ones_v, acc.at[dst_v.at[g * 8 + j]], sem,
                             add=True)
        for _ in range(8):
            pltpu.make_async_copy(ones_v, acc.at[dst_v.at[0]], sem).wait()
        return carry

    lax.fori_loop(0, rows_per_worker // 8, grp, 0)
    plsc.subcore_barrier()

    @pl.when(c == 0)
    def _():
        pltpu.sync_copy(acc.at[pl.ds(s * RPS, RPS)],
                        deg0.at[pl.ds(s * RPS, RPS)])

    @pl.when(c == 1)
    def _():
        pltpu.sync_copy(acc.at[pl.ds(s * RPS, RPS)],
                        deg1.at[pl.ds(s * RPS, RPS)])


def _deg_call(dstm, z128, o128, rows_all):
    rows_per_worker = rows_all // (NC * NS)
    body = functools.partial(_deg_body, rows_per_worker)
    f = pl.kernel(
        body,
        out_type=(
            jax.ShapeDtypeStruct((N_PAD, FH), jnp.float32),
            jax.ShapeDtypeStruct((N_PAD, FH), jnp.float32),
        ),
        mesh=plsc.VectorSubcoreMesh(core_axis_name="c", subcore_axis_name="s"),
        scratch_types=(
            pltpu.VMEM_SHARED((N_PAD, FH), jnp.float32),
            pltpu.VMEM((rows_per_worker, CHUNK), jnp.int32),
            pltpu.VMEM((CHUNK, FH), jnp.float32),
            pltpu.SemaphoreType.DMA,
        ),
    )
    return f(dstm, z128, o128)


CHUNK_A = 64
NBUF = 4
SEG_A = 40


def _agg_body(rows_per_sub, cs0, cs1, srcm, dstm, z128, agg0, agg1,
              acc, src_v, dst_v, r0, r1, r2, r3,
              g0, g1, g2, g3, t0, t1, t2, t3):
    c = lax.axis_index("c")
    s = lax.axis_index("s")
    pltpu.sync_copy(z128, acc.at[pl.ds(s * RPS, RPS)])
    base = s * rows_per_sub
    plsc.subcore_barrier()

    rows = (r0, r1, r2, r3)
    sem_g = (g0, g1, g2, g3)
    sem_s = (t0, t1, t2, t3)

    def run(cs_hbm, agg_hbm):
        def gather(b, k):
            pltpu.async_copy(cs_hbm.at[src_v.at[k]], rows[b], sem_g[b])

        def wait_gather(b):
            pltpu.make_async_copy(cs_hbm.at[src_v.at[0]], rows[b],
                                  sem_g[b]).wait()

        def scatter(b, k):
            pltpu.async_copy(rows[b], acc.at[dst_v.at[k]], sem_s[b],
                             add=True)

        def wait_scatter(b):
            pltpu.make_async_copy(rows[b], acc.at[dst_v.at[0]],
                                  sem_s[b]).wait()

        def seg_body(gidx, carry):
            pltpu.sync_copy(srcm.at[pl.ds(base + gidx * SEG_A, SEG_A)],
                            src_v)
            pltpu.sync_copy(dstm.at[pl.ds(base + gidx * SEG_A, SEG_A)],
                            dst_v)
            gather(0, 0)
            gather(1, 1)
            gather(2, 2)

            def quad(q, carry2):
                for b in range(NBUF):
                    k = NBUF * q + b
                    bprev = (b - 1) % NBUF
                    wait_gather(b)
                    scatter(b, k)

                    @pl.when(k >= 1)
                    def _(bp=bprev):
                        wait_scatter(bp)

                    @pl.when(k + 3 < SEG_A)
                    def _(bp=bprev, kk=k):
                        gather(bp, kk + 3)

                return carry2

            lax.fori_loop(0, SEG_A // NBUF, quad, 0)
            wait_scatter(NBUF - 1)
            return carry

        lax.fori_loop(0, rows_per_sub // SEG_A, seg_body, 0)
        plsc.subcore_barrier()
        pltpu.sync_copy(acc.at[pl.ds(s * RPS, RPS)],
                        agg_hbm.at[pl.ds(s * RPS, RPS)])

    @pl.when(c == 0)
    def _():
        run(cs0, agg0)

    @pl.when(c == 1)
    def _():
        run(cs1, agg1)


def _agg_call(cs0, cs1, srcm, dstm, z128, rows_all64):
    rows_per_sub = rows_all64 // NS
    body = functools.partial(_agg_body, rows_per_sub)
    f = pl.kernel(
        body,
        out_type=(
            jax.ShapeDtypeStruct((N_PAD, FH), jnp.float32),
            jax.ShapeDtypeStruct((N_PAD, FH), jnp.float32),
        ),
        mesh=plsc.VectorSubcoreMesh(core_axis_name="c", subcore_axis_name="s"),
        scratch_types=(
            pltpu.VMEM_SHARED((N_PAD, FH), jnp.float32),
            pltpu.VMEM((SEG_A, CHUNK_A), jnp.int32),
            pltpu.VMEM((SEG_A, CHUNK_A), jnp.int32),
            pltpu.VMEM((CHUNK_A, FH), jnp.float32),
            pltpu.VMEM((CHUNK_A, FH), jnp.float32),
            pltpu.VMEM((CHUNK_A, FH), jnp.float32),
            pltpu.VMEM((CHUNK_A, FH), jnp.float32),
            pltpu.SemaphoreType.DMA,
            pltpu.SemaphoreType.DMA,
            pltpu.SemaphoreType.DMA,
            pltpu.SemaphoreType.DMA,
            pltpu.SemaphoreType.DMA,
            pltpu.SemaphoreType.DMA,
            pltpu.SemaphoreType.DMA,
            pltpu.SemaphoreType.DMA,
        ),
    )
    return f(cs0, cs1, srcm, dstm, z128)


def _scale_body(x_ref, h_ref, d0_ref, d1_ref, cs0_ref, cs1_ref):
    deg = 1.0 + d0_ref[:, 0:1] + d1_ref[:, 0:1]
    dinv = lax.rsqrt(deg)
    cs0_ref[...] = x_ref[...] * dinv
    cs1_ref[...] = h_ref[...] * dinv


def _scale_call(x, h, deg0, deg1):
    B = 400
    grid = N // B
    return pl.pallas_call(
        _scale_body,
        grid=(grid,),
        in_specs=[
            pl.BlockSpec((B, FH), lambda i: (i, 0)),
            pl.BlockSpec((B, FH), lambda i: (i, 0)),
            pl.BlockSpec((B, FH), lambda i: (i, 0)),
            pl.BlockSpec((B, FH), lambda i: (i, 0)),
        ],
        out_specs=[
            pl.BlockSpec((B, FH), lambda i: (i, 0)),
            pl.BlockSpec((B, FH), lambda i: (i, 0)),
        ],
        out_shape=[
            jax.ShapeDtypeStruct((N, FH), jnp.float32),
            jax.ShapeDtypeStruct((N, FH), jnp.float32),
        ],
    )(x, h, deg0, deg1)


def _gate_body(a0_ref, a1_ref, cs0_ref, cs1_ref, d0_ref, d1_ref, c_ref,
               w_ref, b_ref, h_out, c_out):
    dinv = lax.rsqrt(1.0 + d0_ref[:, 0:1] + d1_ref[:, 0:1])
    z0 = (a0_ref[...] + cs0_ref[...]) * dinv
    z1 = (a1_ref[...] + cs1_ref[...]) * dinv
    z = jnp.concatenate([z0, z1], axis=1)
    g4 = jnp.dot(z, w_ref[...], preferred_element_type=jnp.float32)
    g4 = g4 + b_ref[...]
    gi = jax.nn.sigmoid(g4[:, 0 * FH:1 * FH])
    gf = jax.nn.sigmoid(g4[:, 1 * FH:2 * FH])
    go = jax.nn.sigmoid(g4[:, 2 * FH:3 * FH])
    gg = jnp.tanh(g4[:, 3 * FH:4 * FH])
    cn = gf * c_ref[...] + gi * gg
    h_out[...] = go * jnp.tanh(cn)
    c_out[...] = cn


def _gate_call(agg0, agg1, cs0, cs1, deg0, deg1, c, w4, b4):
    B = 400
    grid = N // B
    return pl.pallas_call(
        _gate_body,
        grid=(grid,),
        in_specs=[
            pl.BlockSpec((B, FH), lambda i: (i, 0)),
            pl.BlockSpec((B, FH), lambda i: (i, 0)),
            pl.BlockSpec((B, FH), lambda i: (i, 0)),
            pl.BlockSpec((B, FH), lambda i: (i, 0)),
            pl.BlockSpec((B, FH), lambda i: (i, 0)),
            pl.BlockSpec((B, FH), lambda i: (i, 0)),
            pl.BlockSpec((B, FH), lambda i: (i, 0)),
            pl.BlockSpec((2 * FH, 4 * FH), lambda i: (0, 0)),
            pl.BlockSpec((1, 4 * FH), lambda i: (0, 0)),
        ],
        out_specs=[
            pl.BlockSpec((B, FH), lambda i: (i, 0)),
            pl.BlockSpec((B, FH), lambda i: (i, 0)),
        ],
        out_shape=[
            jax.ShapeDtypeStruct((N, FH), jnp.float32),
            jax.ShapeDtypeStruct((N, FH), jnp.float32),
        ],
    )(agg0, agg1, cs0, cs1, deg0, deg1, c, w4, b4)


def kernel(x, h, c, edge_index, W_i, b_i, W_f, b_f, W_o, b_o, W_g, b_g):
    e = edge_index.shape[1]
    group = NC * NS * CHUNK * 8
    e_pad = ((e + group - 1) // group) * group
    rows_all = e_pad // CHUNK

    src = edge_index[0].astype(jnp.int32)
    dst = edge_index[1].astype(jnp.int32)
    pad = e_pad - e
    src_p = jnp.concatenate([src, jnp.zeros((pad,), jnp.int32)])
    dst_p = jnp.concatenate([dst, jnp.full((pad,), TRASH, jnp.int32)])
    dstm = dst_p.reshape(rows_all, CHUNK)
    srcm64 = src_p.reshape(e_pad // CHUNK_A, CHUNK_A)
    dstm64 = dst_p.reshape(e_pad // CHUNK_A, CHUNK_A)

    z128 = jnp.zeros((RPS, FH), jnp.float32)
    o128 = jnp.ones((CHUNK, FH), jnp.float32)

    deg0, deg1 = _deg_call(dstm, z128, o128, rows_all)
    cs0, cs1 = _scale_call(x, h, deg0, deg1)
    agg0, agg1 = _agg_call(cs0, cs1, srcm64, dstm64, z128, e_pad // CHUNK_A)

    w4 = jnp.concatenate([W_i, W_f, W_o, W_g], axis=1)
    b4 = jnp.concatenate([b_i, b_f, b_o, b_g]).reshape(1, 4 * FH)
    h_new, c_new = _gate_call(agg0, agg1, cs0, cs1, deg0, deg1, c, w4, b4)
    return (h_new, c_new)
